# baseline (speedup 1.0000x reference)
import jax
import jax.numpy as jnp
from jax import lax
from jax.experimental import pallas as pl
from jax.experimental.pallas import tpu as pltpu
from jax.experimental.pallas import tpu_sc as plsc

MAX_POSITIONS = 1000000
EMBED_DIM = 32
BATCH = 16384
HIST = 200

NW = 32
ROWS_W = BATCH // NW
RB = 8
CHUNKS = ROWS_W // RB
NR = 2
NI = 4


def _sc_gather(idx_hbm, table_hbm, out_hbm, idx_v0, idx_v1, idx_v2,
               idx_v3, rows_v0, rows_v1, isem0, isem1, isem2, isem3,
               gsem0, gsem1, wsem0, wsem1):
    idx_v = [idx_v0, idx_v1, idx_v2, idx_v3]
    rows_v = [rows_v0, rows_v1]
    isem = [isem0, isem1, isem2, isem3]
    gsem = [gsem0, gsem1]
    wsem = [wsem0, wsem1]

    wid = lax.axis_index("s") * 2 + lax.axis_index("c")
    row0 = wid * ROWS_W

    def start_idx(c, i):
        pltpu.async_copy(
            idx_hbm.at[pl.ds(row0 + c * RB, RB)], idx_v[i], isem[i])

    def wait_idx(i):
        pltpu.make_async_copy(
            idx_hbm.at[pl.ds(row0, RB)], idx_v[i], isem[i]).wait()

    def fire_gather(i, b):
        for j in range(RB):
            pltpu.async_copy(
                table_hbm.at[idx_v[i].at[j]], rows_v[b].at[j], gsem[b])

    def wait_gather(i, b):
        for j in range(RB):
            pltpu.make_async_copy(
                table_hbm.at[idx_v[i].at[j]], rows_v[b].at[j],
                gsem[b]).wait()

    def start_write(c, b):
        pltpu.async_copy(
            rows_v[b], out_hbm.at[pl.ds(row0 + c * RB, RB)], wsem[b])

    def wait_write(b):
        pltpu.make_async_copy(
            rows_v[b], out_hbm.at[pl.ds(row0, RB)], wsem[b]).wait()

    def step(c, k, wr_wait, prefetch, prev):
        b, i = k % NR, k % NI
        pb, pi = (k - 1) % NR, (k - 1) % NI
        wait_idx(i)
        if wr_wait:
            wait_write(b)
        fire_gather(i, b)
        if prev:
            wait_gather(pi, pb)
            start_write(c - 1, pb)
            if prefetch:
                start_idx(c + NI - 1, pi)

    for i in range(NI):
        start_idx(i, i)
    step(0, 0, wr_wait=False, prefetch=False, prev=False)
    step(1, 1, wr_wait=False, prefetch=True, prev=True)
    step(2, 2, wr_wait=True, prefetch=True, prev=True)
    step(3, 3, wr_wait=True, prefetch=True, prev=True)

    def body(g, carry):
        c0 = g * 4
        for k in range(4):
            step(c0 + k, k, wr_wait=True, prefetch=True, prev=True)
        return carry

    lax.fori_loop(1, (CHUNKS - 4) // 4, body, 0)

    step(60, 0, wr_wait=True, prefetch=True, prev=True)
    step(61, 1, wr_wait=True, prefetch=False, prev=True)
    step(62, 2, wr_wait=True, prefetch=False, prev=True)
    step(63, 3, wr_wait=True, prefetch=False, prev=True)
    wait_gather(63 % NI, 63 % NR)
    start_write(63, 63 % NR)
    for b in range(NR):
        wait_write(b)


@jax.jit
def _lookup(idx, table):
    mesh = plsc.VectorSubcoreMesh(core_axis_name="c", subcore_axis_name="s")
    f = pl.kernel(
        _sc_gather,
        out_type=jax.ShapeDtypeStruct((BATCH, HIST, EMBED_DIM),
                                      jnp.float32),
        mesh=mesh,
        scratch_types=(
            [pltpu.VMEM((RB, HIST), jnp.int32) for _ in range(NI)]
            + [pltpu.VMEM((RB, HIST, EMBED_DIM), jnp.float32)
               for _ in range(NR)]
            + [pltpu.SemaphoreType.DMA for _ in range(NI + 2 * NR)]
        ),
        compiler_params=pltpu.CompilerParams(use_tc_tiling_on_sc=False),
    )
    return f(idx, table)


def kernel(inputs, embeddings):
    return _lookup(inputs.astype(jnp.int32), embeddings)
